# baseline (speedup 1.0000x reference)
import functools

import jax
import jax.numpy as jnp
from jax import lax
from jax.experimental import pallas as pl
from jax.experimental.pallas import tpu as pltpu
from jax.experimental.pallas import tpu_sc as plsc

_SB = 0.1
_N = 65536
_NBINS = 512
_NS = 16
_CHUNK = _N // _NS
_VECS = _CHUNK // 16
_ROW = 640

_mesh = plsc.VectorSubcoreMesh(core_axis_name="c", subcore_axis_name="s")


@functools.partial(
    pl.kernel,
    mesh=_mesh,
    compiler_params=pltpu.CompilerParams(needs_layout_passes=False),
    out_type=jax.ShapeDtypeStruct((2, 16), jnp.float32),
    scratch_types=[
        pltpu.VMEM((_CHUNK,), jnp.float32),
        pltpu.VMEM((_CHUNK,), jnp.int32),
        pltpu.VMEM((16 * _NBINS,), jnp.float32),
        pltpu.VMEM((_ROW,), jnp.float32),
        pltpu.VMEM_SHARED((_NS, _ROW), jnp.float32),
        pltpu.VMEM((_NS * _ROW,), jnp.float32),
        pltpu.VMEM((16,), jnp.float32),
    ],
)
def _sc_loss(beta_hbm, pid_hbm, out_hbm, beta_v, pid_v, bins_v, red_v,
             sh_mx, fin_v, out_v):
    cid = lax.axis_index("c")
    sid = lax.axis_index("s")
    base = sid * _CHUNK
    pltpu.sync_copy(beta_hbm.at[pl.ds(base, _CHUNK)], beta_v)
    pltpu.sync_copy(pid_hbm.at[pl.ds(base, _CHUNK)], pid_v)

    lane = lax.broadcasted_iota(jnp.int32, (16,), 0)
    neg = jnp.full((16,), -1.0, jnp.float32)
    zero = jnp.zeros((16,), jnp.float32)

    def init_body(i, carry):
        bins_v[pl.ds(i * 16, 16)] = neg
        return carry

    lax.fori_loop(0, 16 * _NBINS // 16, init_body, 0)

    lane_base = lane * _NBINS

    def body(j, carry):
        nsum, ncnt = carry
        pidv = pid_v[pl.ds(j * 16, 16)]
        betav = beta_v[pl.ds(j * 16, 16)]
        flat = lane_base + pidv
        cur = plsc.load_gather(bins_v, [flat])
        plsc.store_scatter(bins_v, [flat], jnp.maximum(cur, betav))
        isnz = pidv == 0
        nsum = nsum + jnp.where(isnz, betav, 0.0)
        ncnt = ncnt + jnp.where(isnz, 1.0, 0.0)
        return nsum, ncnt

    nsum, ncnt = lax.fori_loop(0, _VECS, body, (zero, zero))

    def red_body(c, carry):
        acc = bins_v[pl.ds(c * 16, 16)]

        def rrow(l, a):
            return jnp.maximum(a, bins_v[pl.ds(l * _NBINS + c * 16, 16)])

        acc = lax.fori_loop(1, 16, rrow, acc)
        red_v[pl.ds(c * 16, 16)] = acc
        return carry

    lax.fori_loop(0, _NBINS // 16, red_body, 0)

    red_v[pl.ds(_NBINS, 16)] = nsum
    red_v[pl.ds(_NBINS + 16, 16)] = ncnt

    pltpu.sync_copy(red_v, sh_mx.at[sid])
    plsc.subcore_barrier()

    @pl.when(sid == 0)
    def _merge():
        for w in range(_NS):
            pltpu.sync_copy(sh_mx.at[w], fin_v.at[pl.ds(w * _ROW, _ROW)])

        def fin_body(c, carry):
            ssum, scnt = carry
            acc = fin_v[pl.ds(c * 16, 16)]

            def rrow(w, a):
                return jnp.maximum(a, fin_v[pl.ds(w * _ROW + c * 16, 16)])

            acc = lax.fori_loop(1, _NS, rrow, acc)
            gbin = c * 16 + lane
            pres = jnp.logical_and(acc >= 0.0, gbin > 0)
            ssum = ssum + jnp.where(pres, 1.0 - acc, 0.0)
            scnt = scnt + jnp.where(pres, 1.0, 0.0)
            return ssum, scnt

        ssum, scnt = lax.fori_loop(0, _NBINS // 16, fin_body, (zero, zero))

        def nzred(w, carry):
            s, c = carry
            s = s + fin_v[pl.ds(w * _ROW + _NBINS, 16)]
            c = c + fin_v[pl.ds(w * _ROW + _NBINS + 16, 16)]
            return s, c

        nsum_t, ncnt_t = lax.fori_loop(0, _NS, nzred, (zero, zero))

        ssum_v = jnp.full((16,), jnp.sum(ssum), jnp.float32)
        scnt_v = jnp.full((16,), jnp.sum(scnt), jnp.float32)
        ns_v = jnp.full((16,), jnp.sum(nsum_t), jnp.float32)
        nc_v = jnp.full((16,), jnp.sum(ncnt_t), jnp.float32)
        loss_v = ssum_v / scnt_v
        noise_v = jnp.where(nc_v > 0.0, _SB * ns_v / jnp.maximum(nc_v, 1.0), 0.0)
        out_v[...] = loss_v + noise_v
        pltpu.sync_copy(out_v, out_hbm.at[cid])


@jax.jit
def kernel(beta, particle_id, ec_hit_mask):
    pid = jnp.where(ec_hit_mask, particle_id, 0).astype(jnp.int32)
    return _sc_loss(beta, pid)[0, 0]
